# baseline (speedup 1.0000x reference)
import functools

import jax
import jax.numpy as jnp
from jax import lax
from jax.experimental import pallas as pl
from jax.experimental.pallas import tpu as pltpu
from jax.experimental.pallas import tpu_sc as plsc

_ENT = 1000000
_REL = 1000
_DIM = 64
_BATCH = 16384
_NC = 2
_NS = 16
_NW = _NC * _NS
_BPW = _BATCH // _NW
_HALF = _BPW // 2
_CH = 128
_TW = 512
_SPLIT = 500224


def _rsqrt16(s):
    i = plsc.bitcast(s, jnp.int32)
    i = jnp.int32(0x5F3759DF) - lax.shift_right_logical(i, 1)
    y = plsc.bitcast(i, jnp.float32)
    half = s * 0.5
    for _ in range(3):
        y = y * (1.5 - half * y * y)
    return y


def _pair_transpose_body(a_ref, b_ref, o_ref):
    eye = jnp.eye(_DIM, dtype=jnp.float32)
    dn = (((0,), (0,)), ((), ()))
    at = lax.dot_general(a_ref[...], eye, dn,
                         preferred_element_type=jnp.float32)
    bt = lax.dot_general(b_ref[...], eye, dn,
                         preferred_element_type=jnp.float32)
    o_ref[...] = jnp.concatenate([at, bt], axis=1)


def _pair_transpose(ent_t):
    grid = _SPLIT // _TW
    return pl.pallas_call(
        _pair_transpose_body,
        grid=(grid,),
        in_specs=[
            pl.BlockSpec((_DIM, _TW), lambda i: (0, i)),
            pl.BlockSpec((_DIM, _TW), lambda i: (0, i + _SPLIT // _TW)),
        ],
        out_specs=pl.BlockSpec((_TW, 2 * _DIM), lambda i: (i, 0)),
        out_shape=jax.ShapeDtypeStruct((_SPLIT, 2 * _DIM), jnp.float32),
    )(ent_t, ent_t)


def kernel(ent_embeddings, rel_embeddings, batch_h, batch_t, batch_r):
    ent2 = _pair_transpose(ent_embeddings.T)
    rel2 = rel_embeddings.reshape(_REL // 2, 2 * _DIM)
    mesh = plsc.VectorSubcoreMesh(core_axis_name="c", subcore_axis_name="s")

    @functools.partial(
        pl.kernel,
        out_type=jax.ShapeDtypeStruct((_BATCH,), jnp.float32),
        mesh=mesh,
        compiler_params=pltpu.CompilerParams(
            needs_layout_passes=False, use_tc_tiling_on_sc=True),
        scratch_types=[
            pltpu.VMEM((_BPW,), jnp.int32),
            pltpu.VMEM((_BPW,), jnp.int32),
            pltpu.VMEM((_BPW,), jnp.int32),
            pltpu.VMEM((_BPW,), jnp.int32),
            pltpu.VMEM((_BPW,), jnp.int32),
            pltpu.VMEM((_BPW,), jnp.int32),
            pltpu.VMEM((_HALF, 2 * _DIM), jnp.float32),
            pltpu.VMEM((_HALF, 2 * _DIM), jnp.float32),
            pltpu.VMEM((_HALF, 2 * _DIM), jnp.float32),
            pltpu.VMEM((_BPW,), jnp.float32),
            pltpu.SemaphoreType.DMA,
        ],
    )
    def k(ent_hbm, rel_hbm, bh_hbm, bt_hbm, br_hbm, out_hbm,
          ih_v, it_v, ir_v, ph_v, pt_v, pr_v, h_v, t_v, r_v, o_v, sem):
        wid = lax.axis_index("s") * _NC + lax.axis_index("c")
        base = wid * _BPW

        pltpu.sync_copy(bh_hbm.at[pl.ds(base, _BPW)], ih_v)
        pltpu.sync_copy(bt_hbm.at[pl.ds(base, _BPW)], it_v)
        pltpu.sync_copy(br_hbm.at[pl.ds(base, _BPW)], ir_v)

        def split(i, _):
            sl = pl.ds(i * 16, 16)
            for iv, pv in ((ih_v, ph_v), (it_v, pt_v)):
                e = iv[sl]
                ge = (e >= jnp.int32(_SPLIT)).astype(jnp.int32)
                pv[sl] = ge * 64
                iv[sl] = e - ge * jnp.int32(_SPLIT)
            er = ir_v[sl]
            pr_v[sl] = lax.bitwise_and(er, jnp.int32(1)) * 64
            ir_v[sl] = lax.shift_right_logical(er, 1)
            return 0

        lax.fori_loop(0, _BPW // 16, split, 0)

        def half(hb):
            off = hb * _HALF
            copies = []
            for c in range(_HALF // _CH):
                s_idx = pl.ds(off + c * _CH, _CH)
                d_idx = pl.ds(c * _CH, _CH)
                copies.append(pltpu.async_copy(
                    ent_hbm.at[ih_v.at[s_idx]], h_v.at[d_idx], sem))
                copies.append(pltpu.async_copy(
                    ent_hbm.at[it_v.at[s_idx]], t_v.at[d_idx], sem))
                copies.append(pltpu.async_copy(
                    rel_hbm.at[ir_v.at[s_idx]], r_v.at[d_idx], sem))
            for cp in copies:
                cp.wait()

            def group(g, _):
                gb = g * 16
                p16h = ph_v[pl.ds(off + gb, 16)]
                p16t = pt_v[pl.ds(off + gb, 16)]
                p16r = pr_v[pl.ds(off + gb, 16)]
                for jj in range(16):
                    oh = p16h[jj]
                    ot = p16t[jj]
                    orr = p16r[jj]
                    sh = jnp.zeros((16,), jnp.float32)
                    st = jnp.zeros((16,), jnp.float32)
                    sr = jnp.zeros((16,), jnp.float32)
                    hs, ts, rs = [], [], []
                    for kk in range(_DIM // 16):
                        hv = h_v[gb + jj, pl.ds(oh + kk * 16, 16)]
                        tv = t_v[gb + jj, pl.ds(ot + kk * 16, 16)]
                        rv = r_v[gb + jj, pl.ds(orr + kk * 16, 16)]
                        hs.append(hv)
                        ts.append(tv)
                        rs.append(rv)
                        sh = sh + hv * hv
                        st = st + tv * tv
                        sr = sr + rv * rv
                    eps = jnp.float32(1e-24)
                    ih = _rsqrt16(jnp.full(
                        (16,), jnp.maximum(jnp.sum(sh), eps)))
                    it = _rsqrt16(jnp.full(
                        (16,), jnp.maximum(jnp.sum(st), eps)))
                    ir = _rsqrt16(jnp.full(
                        (16,), jnp.maximum(jnp.sum(sr), eps)))
                    acc = jnp.zeros((16,), jnp.float32)
                    for kk in range(_DIM // 16):
                        acc = acc + jnp.abs(hs[kk] * ih + rs[kk] * ir
                                            - ts[kk] * it)
                    lane = lax.iota(jnp.int32, 16)
                    plsc.store_scatter(
                        o_v, [jnp.full((16,), off + gb + jj, jnp.int32)],
                        plsc.cumsum(acc), mask=lane == 15)
                return 0

            lax.fori_loop(0, _HALF // 16, group, 0)

        half(0)
        half(1)

        pltpu.sync_copy(o_v, out_hbm.at[pl.ds(base, _BPW)])

    return k(ent2, rel2, batch_h, batch_t, batch_r)


# speedup vs baseline: 2.3471x; 2.3471x over previous
import functools

import jax
import jax.numpy as jnp
from jax import lax
from jax.experimental import pallas as pl
from jax.experimental.pallas import tpu as pltpu
from jax.experimental.pallas import tpu_sc as plsc

_ENT = 1000000
_REL = 1000
_DIM = 64
_BATCH = 16384
_NC = 2
_NS = 16
_NW = _NC * _NS
_BPW = _BATCH // _NW
_CHN = 16
_NCHUNK = _BPW // _CHN


def _rsqrt16(s):
    i = plsc.bitcast(s, jnp.int32)
    i = jnp.int32(0x5F3759DF) - lax.shift_right_logical(i, 1)
    y = plsc.bitcast(i, jnp.float32)
    half = s * 0.5
    for _ in range(3):
        y = y * (1.5 - half * y * y)
    return y


def kernel(ent_embeddings, rel_embeddings, batch_h, batch_t, batch_r):
    rel2 = rel_embeddings.reshape(_REL // 2, 2 * _DIM)
    ent3 = ent_embeddings.reshape(_ENT // 8, 8, _DIM)
    mesh = plsc.VectorSubcoreMesh(core_axis_name="c", subcore_axis_name="s")

    @functools.partial(
        pl.kernel,
        out_type=jax.ShapeDtypeStruct((_BATCH,), jnp.float32),
        mesh=mesh,
        compiler_params=pltpu.CompilerParams(
            needs_layout_passes=False, use_tc_tiling_on_sc=True),
        scratch_types=[
            pltpu.VMEM((_BPW,), jnp.int32),
            pltpu.VMEM((_BPW,), jnp.int32),
            pltpu.VMEM((_BPW,), jnp.int32),
            pltpu.VMEM((_BPW,), jnp.int32),
            pltpu.VMEM((_BPW,), jnp.int32),
            pltpu.VMEM((_BPW,), jnp.int32),
            pltpu.VMEM((_CHN, 8, _DIM), jnp.float32),
            pltpu.VMEM((_CHN, 8, _DIM), jnp.float32),
            pltpu.VMEM((_CHN, 2 * _DIM), jnp.float32),
            pltpu.VMEM((_BPW,), jnp.float32),
            pltpu.SemaphoreType.DMA,
        ],
    )
    def k(ent_hbm, rel_hbm, bh_hbm, bt_hbm, br_hbm, out_hbm,
          ihb_v, itb_v, ihr_v, itr_v, ir_v, pr_v, h_v, t_v, r_v, o_v, sem):
        wid = lax.axis_index("s") * _NC + lax.axis_index("c")
        base = wid * _BPW

        pltpu.sync_copy(bh_hbm.at[pl.ds(base, _BPW)], ihb_v)
        pltpu.sync_copy(bt_hbm.at[pl.ds(base, _BPW)], itb_v)
        pltpu.sync_copy(br_hbm.at[pl.ds(base, _BPW)], ir_v)

        def split(i, _):
            sl = pl.ds(i * 16, 16)
            eh = ihb_v[sl]
            et = itb_v[sl]
            er = ir_v[sl]
            ihr_v[sl] = lax.bitwise_and(eh, jnp.int32(7))
            itr_v[sl] = lax.bitwise_and(et, jnp.int32(7))
            ihb_v[sl] = lax.shift_right_logical(eh, 3)
            itb_v[sl] = lax.shift_right_logical(et, 3)
            pr_v[sl] = lax.bitwise_and(er, jnp.int32(1)) * 64
            ir_v[sl] = lax.shift_right_logical(er, 1)
            return 0

        lax.fori_loop(0, _BPW // 16, split, 0)

        def chunk(c, _):
            cb = c * _CHN
            sl = pl.ds(cb, _CHN)
            copies = [
                pltpu.async_copy(rel_hbm.at[ir_v.at[sl]], r_v, sem),
            ]
            bh16 = ihb_v[sl]
            bt16 = itb_v[sl]
            for j in range(_CHN):
                copies.append(pltpu.async_copy(
                    ent_hbm.at[bh16[j]], h_v.at[j], sem))
                copies.append(pltpu.async_copy(
                    ent_hbm.at[bt16[j]], t_v.at[j], sem))
            for cp in copies:
                cp.wait()

            rh16 = ihr_v[sl]
            rt16 = itr_v[sl]
            pr16 = pr_v[sl]
            for jj in range(_CHN):
                rh = rh16[jj]
                rt = rt16[jj]
                orr = pr16[jj]
                sh = jnp.zeros((16,), jnp.float32)
                st = jnp.zeros((16,), jnp.float32)
                sr = jnp.zeros((16,), jnp.float32)
                hs, ts, rs = [], [], []
                for kk in range(_DIM // 16):
                    hv = h_v[jj, rh, pl.ds(kk * 16, 16)]
                    tv = t_v[jj, rt, pl.ds(kk * 16, 16)]
                    rv = r_v[jj, pl.ds(orr + kk * 16, 16)]
                    hs.append(hv)
                    ts.append(tv)
                    rs.append(rv)
                    sh = sh + hv * hv
                    st = st + tv * tv
                    sr = sr + rv * rv
                eps = jnp.float32(1e-24)
                ih = _rsqrt16(jnp.full((16,), jnp.maximum(jnp.sum(sh), eps)))
                it = _rsqrt16(jnp.full((16,), jnp.maximum(jnp.sum(st), eps)))
                ir = _rsqrt16(jnp.full((16,), jnp.maximum(jnp.sum(sr), eps)))
                acc = jnp.zeros((16,), jnp.float32)
                for kk in range(_DIM // 16):
                    acc = acc + jnp.abs(hs[kk] * ih + rs[kk] * ir
                                        - ts[kk] * it)
                lane = lax.iota(jnp.int32, 16)
                plsc.store_scatter(
                    o_v, [jnp.full((16,), cb + jj, jnp.int32)],
                    plsc.cumsum(acc), mask=lane == 15)
            return 0

        lax.fori_loop(0, _NCHUNK, chunk, 0)

        pltpu.sync_copy(o_v, out_hbm.at[pl.ds(base, _BPW)])

    return k(ent3, rel2, batch_h, batch_t, batch_r)
